# baseline (speedup 1.0000x reference)
import functools
import math

import jax
import jax.numpy as jnp
from jax import lax
from jax.experimental import pallas as pl
from jax.experimental.pallas import tpu as pltpu
from jax.experimental.pallas import tpu_sc as plsc

VOCAB = 1000000
D_MODEL = 64
LANES = 16
NUM_CORES = 2
NUM_SUBCORES = 16
NUM_WORKERS = NUM_CORES * NUM_SUBCORES
SCALE = math.sqrt(D_MODEL)

NCOLS = VOCAB // 128
COLS_PER_W = NCOLS // NUM_WORKERS
K1_SLOTS = 2

CHUNK = 128
NSLOTS = 2
OSLOTS = 2

_sc_params = pltpu.CompilerParams(
    use_tc_tiling_on_sc=True, needs_layout_passes=False)
_mesh = plsc.VectorSubcoreMesh(core_axis_name="c", subcore_axis_name="s")


@functools.partial(
    pl.kernel,
    mesh=_mesh,
    out_type=jax.ShapeDtypeStruct((VOCAB // 2, 128), jnp.float32),
    scratch_types=[
        pltpu.VMEM((K1_SLOTS, D_MODEL, 131), jnp.float32),
        pltpu.VMEM((K1_SLOTS, D_MODEL, 128), jnp.float32),
        pltpu.SemaphoreType.DMA,
        pltpu.SemaphoreType.DMA,
    ],
    compiler_params=_sc_params,
)
def _k1_convert(wT_hbm, tail_hbm, t2_hbm, src_v, dst_v, rsem, wsem):
    wid = lax.axis_index("s") * NUM_CORES + lax.axis_index("c")

    def read(i, slot):
        pltpu.async_copy(
            wT_hbm.at[:, pl.ds((wid + i * NUM_WORKERS) * 128, 128)],
            src_v.at[slot, :, pl.ds(0, 128)], rsem)

    def wait_read(i, slot):
        pltpu.make_async_copy(
            wT_hbm.at[:, pl.ds((wid + i * NUM_WORKERS) * 128, 128)],
            src_v.at[slot, :, pl.ds(0, 128)], rsem).wait()

    def write(i, slot):
        pltpu.async_copy(
            dst_v.at[slot],
            t2_hbm.at[pl.ds((wid + i * NUM_WORKERS) * 64, 64)], wsem)

    def wait_write(i, slot):
        pltpu.make_async_copy(
            dst_v.at[slot],
            t2_hbm.at[pl.ds((wid + i * NUM_WORKERS) * 64, 64)], wsem).wait()

    def transpose_scale(slot):
        iota = lax.iota(jnp.int32, LANES)
        src = src_v.at[slot]

        @plsc.parallel_loop(0, 128, unroll=8)
        def _c_loop(c):
            c16 = jnp.full((LANES,), 0, jnp.int32) + c
            u = lax.shift_right_logical(c, 1)
            half = lax.bitwise_and(c, 1) * D_MODEL
            for k in range(D_MODEL // LANES):
                g = plsc.load_gather(src, [iota + (k * LANES), c16])
                dst_v[slot, u, pl.ds(half + k * LANES, LANES)] = g * SCALE

    read(0, 0)
    read(1, 1)

    def body(i, carry):
        slot = lax.rem(i, K1_SLOTS)
        wait_read(i, slot)

        @pl.when(i >= K1_SLOTS)
        def _drain():
            wait_write(i, slot)

        transpose_scale(slot)
        write(i, slot)

        @pl.when(i + K1_SLOTS < COLS_PER_W)
        def _prefetch():
            read(i + K1_SLOTS, lax.rem(i + K1_SLOTS, K1_SLOTS))
        return carry

    lax.fori_loop(0, COLS_PER_W, body, 0)
    for _ in range(min(K1_SLOTS, COLS_PER_W)):
        wait_write(0, 0)

    @pl.when(wid < NCOLS - COLS_PER_W * NUM_WORKERS)
    def _extra():
        pltpu.sync_copy(
            wT_hbm.at[:, pl.ds((COLS_PER_W * NUM_WORKERS + wid) * 128, 128)],
            src_v.at[0, :, pl.ds(0, 128)])
        transpose_scale(0)
        pltpu.sync_copy(
            dst_v.at[0],
            t2_hbm.at[pl.ds((COLS_PER_W * NUM_WORKERS + wid) * 64, 64)])

    @pl.when(wid == NUM_WORKERS - 1)
    def _tail():
        pltpu.sync_copy(tail_hbm, dst_v.at[0, pl.ds(0, 32)])
        pltpu.sync_copy(dst_v.at[0, pl.ds(0, 32)],
                        t2_hbm.at[pl.ds(VOCAB // 2 - 32, 32)])


def _k2_gather(xT, t2, batch, hist):
    n_blocks = batch // CHUNK

    @functools.partial(
        pl.kernel,
        mesh=_mesh,
        out_type=jax.ShapeDtypeStruct(
            (hist, 8, n_blocks, 8, 128), jnp.float32),
        scratch_types=[
            pltpu.VMEM((hist, CHUNK), jnp.int32),
            pltpu.VMEM((NSLOTS, CHUNK), jnp.int32),
            pltpu.VMEM((NSLOTS, CHUNK, 129), jnp.float32),
            pltpu.VMEM((OSLOTS, 8, 8, 128), jnp.float32),
            pltpu.SemaphoreType.DMA,
            pltpu.SemaphoreType.DMA,
        ],
        compiler_params=_sc_params,
    )
    def k(xT_hbm, t2_hbm, out_hbm, idx_v, u_v, rows_v, t5_v, gsem, osem):
        wid = lax.axis_index("s") * NUM_CORES + lax.axis_index("c")
        pltpu.sync_copy(xT_hbm.at[:, pl.ds(wid * CHUNK, CHUNK)], idx_v)

        def make_u(h, slot):
            for m in range(CHUNK // LANES):
                sl = pl.ds(m * LANES, LANES)
                u_v[slot, sl] = lax.shift_right_logical(idx_v[h, sl], 1)

        def gather(slot):
            return pltpu.async_copy(
                t2_hbm.at[u_v.at[slot]],
                rows_v.at[slot, :, pl.ds(0, 128)], gsem)

        def wait_gather(slot):
            pltpu.make_async_copy(
                t2_hbm.at[u_v.at[slot]],
                rows_v.at[slot, :, pl.ds(0, 128)], gsem).wait()

        make_u(0, 0)
        gather(0)
        make_u(1, 1)
        gather(1)

        def transpose_select(h, gslot, oslot):
            iota = lax.iota(jnp.int32, LANES)
            src = rows_v.at[gslot]

            @plsc.parallel_loop(0, CHUNK // LANES, unroll=4)
            def _m_loop(m):
                c16 = iota + m * LANES
                p64 = lax.bitwise_and(
                    idx_v[h, pl.ds(m * LANES, LANES)], 1) * D_MODEL
                for d in range(D_MODEL):
                    g = plsc.load_gather(src, [c16, p64 + d])
                    t5_v[oslot, d // 8, d % 8,
                         pl.ds(m * LANES, LANES)] = g

        def body(h, carry):
            gslot = lax.rem(h, NSLOTS)
            oslot = lax.rem(h, OSLOTS)
            wait_gather(gslot)

            @pl.when(h >= OSLOTS)
            def _wait_out():
                pltpu.make_async_copy(
                    t5_v.at[oslot, :, :, pl.ds(0, 128)],
                    out_hbm.at[h, :, wid], osem).wait()

            transpose_select(h, gslot, oslot)
            pltpu.async_copy(
                t5_v.at[oslot, :, :, pl.ds(0, 128)],
                out_hbm.at[h, :, wid], osem)

            @pl.when(h + 2 < hist)
            def _next():
                nslot = lax.rem(h + 2, NSLOTS)
                make_u(h + 2, nslot)
                gather(nslot)
            return carry

        lax.fori_loop(0, hist, body, 0)
        for _ in range(OSLOTS):
            pltpu.make_async_copy(
                t5_v.at[0, :, :, pl.ds(0, 128)],
                out_hbm.at[0, :, wid], osem).wait()

    return k(xT, t2)


def kernel(x, emb_weight):
    batch, hist = x.shape
    tail = emb_weight[VOCAB - 64:].reshape(32, 128) * SCALE
    t2 = _k1_convert(emb_weight.T, tail)
    xT = x.astype(jnp.int32).T
    out5 = _k2_gather(xT, t2, batch, hist)
    return out5.transpose(2, 4, 0, 1, 3).reshape(batch, hist, D_MODEL)


# speedup vs baseline: 2.2218x; 2.2218x over previous
import functools
import math

import jax
import jax.numpy as jnp
from jax import lax
from jax.experimental import pallas as pl
from jax.experimental.pallas import tpu as pltpu
from jax.experimental.pallas import tpu_sc as plsc

D_MODEL = 64
LANES = 16
NUM_CORES = 2
NUM_SUBCORES = 16
NUM_WORKERS = NUM_CORES * NUM_SUBCORES
CHUNK = 128
NSLOTS = 4
OSLOTS = 2
SCALE = math.sqrt(D_MODEL)


def _sc_embed(xT, table, batch, hist):
    n_blocks = batch // CHUNK
    mesh = plsc.VectorSubcoreMesh(core_axis_name="c", subcore_axis_name="s")

    @functools.partial(
        pl.kernel,
        mesh=mesh,
        out_type=jax.ShapeDtypeStruct(
            (hist, 8, n_blocks, 8, 128), jnp.float32),
        scratch_types=[
            pltpu.VMEM((hist, CHUNK), jnp.int32),
            pltpu.VMEM((NSLOTS, CHUNK, D_MODEL), jnp.float32),
            pltpu.VMEM((OSLOTS, 8, 8, 129), jnp.float32),
            pltpu.SemaphoreType.DMA,
            pltpu.SemaphoreType.DMA,
        ],
        compiler_params=pltpu.CompilerParams(
            use_tc_tiling_on_sc=False, needs_layout_passes=False),
    )
    def k(xT_hbm, table_hbm, out_hbm, idx_v, rows_v, t5_v, gsem, osem):
        wid = lax.axis_index("s") * NUM_CORES + lax.axis_index("c")
        pltpu.sync_copy(xT_hbm.at[:, pl.ds(wid * CHUNK, CHUNK)], idx_v)

        def gather(h, slot):
            return pltpu.async_copy(
                table_hbm.at[idx_v.at[h]], rows_v.at[slot], gsem)

        gather(0, 0)
        gather(1, 1)

        def transpose_scale(gslot, oslot):
            iota = lax.iota(jnp.int32, LANES)
            ti_half = jax.lax.shift_right_logical(iota, 3)
            r_vec = jax.lax.bitwise_and(iota, 7)
            dst = t5_v.at[oslot]

            @plsc.parallel_loop(0, CHUNK, unroll=8)
            def _c_loop(c):
                c16 = jnp.full((LANES,), 0, jnp.int32) + c
                for k in range(D_MODEL // LANES):
                    v = rows_v[gslot, c, pl.ds(k * LANES, LANES)]
                    plsc.store_scatter(
                        dst, [ti_half + (2 * k), r_vec, c16], v * SCALE)

        def body(h, carry):
            gslot = lax.rem(h, NSLOTS)
            oslot = lax.rem(h, OSLOTS)
            pltpu.make_async_copy(
                table_hbm.at[idx_v.at[h]], rows_v.at[gslot], gsem).wait()

            @pl.when(h >= OSLOTS)
            def _wait_out():
                pltpu.make_async_copy(
                    t5_v.at[oslot, :, :, pl.ds(0, 128)],
                    out_hbm.at[h, :, wid], osem).wait()

            transpose_scale(gslot, oslot)
            pltpu.async_copy(
                t5_v.at[oslot, :, :, pl.ds(0, 128)],
                out_hbm.at[h, :, wid], osem)

            @pl.when(h + 2 < hist)
            def _next_gather():
                gather(h + 2, lax.rem(h + 2, NSLOTS))
            return carry

        lax.fori_loop(0, hist, body, 0)

        for _ in range(OSLOTS):
            pltpu.make_async_copy(
                t5_v.at[0, :, :, pl.ds(0, 128)],
                out_hbm.at[0, :, wid], osem).wait()

    return k(xT, table)


def kernel(x, emb_weight):
    batch, hist = x.shape
    xT = x.astype(jnp.int32).T
    out5 = _sc_embed(xT, emb_weight, batch, hist)
    return out5.transpose(2, 4, 0, 1, 3).reshape(batch, hist, D_MODEL)


# speedup vs baseline: 2.3356x; 1.0512x over previous
import functools
import math

import jax
import jax.numpy as jnp
from jax import lax
from jax.experimental import pallas as pl
from jax.experimental.pallas import tpu as pltpu
from jax.experimental.pallas import tpu_sc as plsc

D_MODEL = 64
LANES = 16
NUM_CORES = 2
NUM_SUBCORES = 16
NUM_WORKERS = NUM_CORES * NUM_SUBCORES
CHUNK = 128
NSLOTS = 4
OSLOTS = 2
SCALE = math.sqrt(D_MODEL)


def _sc_embed(xT, table, batch, hist):
    n_blocks = batch // CHUNK
    mesh = plsc.VectorSubcoreMesh(core_axis_name="c", subcore_axis_name="s")

    @functools.partial(
        pl.kernel,
        mesh=mesh,
        out_type=jax.ShapeDtypeStruct(
            (hist, 8, n_blocks, 8, 128), jnp.float32),
        scratch_types=[
            pltpu.VMEM((hist, CHUNK), jnp.int32),
            pltpu.VMEM((NSLOTS, CHUNK, D_MODEL), jnp.float32),
            pltpu.VMEM((OSLOTS, 8, 8, 129), jnp.float32),
            pltpu.SemaphoreType.DMA,
            pltpu.SemaphoreType.DMA,
        ],
        compiler_params=pltpu.CompilerParams(
            use_tc_tiling_on_sc=False, needs_layout_passes=False),
    )
    def k(xT_hbm, table_hbm, out_hbm, idx_v, rows_v, t5_v, gsem, osem):
        wid = lax.axis_index("s") * NUM_CORES + lax.axis_index("c")
        pltpu.sync_copy(xT_hbm.at[:, pl.ds(wid * CHUNK, CHUNK)], idx_v)

        def gather(h, slot):
            return pltpu.async_copy(
                table_hbm.at[idx_v.at[h]], rows_v.at[slot], gsem)

        gather(0, 0)
        gather(1, 1)
        gather(2, 2)

        def transpose_scale(gslot, oslot):
            iota = lax.iota(jnp.int32, LANES)
            ti_half = jax.lax.shift_right_logical(iota, 3)
            r_vec = jax.lax.bitwise_and(iota, 7)
            dst = t5_v.at[oslot]

            @plsc.parallel_loop(0, CHUNK, unroll=8)
            def _c_loop(c):
                c16 = jnp.full((LANES,), 0, jnp.int32) + c
                for k in range(D_MODEL // LANES):
                    v = rows_v[gslot, c, pl.ds(k * LANES, LANES)]
                    plsc.store_scatter(
                        dst, [ti_half + (2 * k), r_vec, c16], v * SCALE)

        def body(h, carry):
            gslot = lax.rem(h, NSLOTS)
            oslot = lax.rem(h, OSLOTS)
            pltpu.make_async_copy(
                table_hbm.at[idx_v.at[h]], rows_v.at[gslot], gsem).wait()

            @pl.when(h >= OSLOTS)
            def _wait_out():
                pltpu.make_async_copy(
                    t5_v.at[oslot, :, :, pl.ds(0, 128)],
                    out_hbm.at[h, :, wid], osem).wait()

            transpose_scale(gslot, oslot)
            pltpu.async_copy(
                t5_v.at[oslot, :, :, pl.ds(0, 128)],
                out_hbm.at[h, :, wid], osem)

            @pl.when(h + 3 < hist)
            def _next_gather():
                gather(h + 3, lax.rem(h + 3, NSLOTS))
            return carry

        lax.fori_loop(0, hist, body, 0)

        for _ in range(OSLOTS):
            pltpu.make_async_copy(
                t5_v.at[0, :, :, pl.ds(0, 128)],
                out_hbm.at[0, :, wid], osem).wait()

    return k(xT, table)


def kernel(x, emb_weight):
    batch, hist = x.shape
    xT = x.astype(jnp.int32).T
    out5 = _sc_embed(xT, emb_weight, batch, hist)
    return out5.transpose(2, 4, 0, 1, 3).reshape(batch, hist, D_MODEL)


# speedup vs baseline: 2.3541x; 1.0079x over previous
import functools
import math

import jax
import jax.numpy as jnp
from jax import lax
from jax.experimental import pallas as pl
from jax.experimental.pallas import tpu as pltpu
from jax.experimental.pallas import tpu_sc as plsc

D_MODEL = 64
LANES = 16
NUM_CORES = 2
NUM_SUBCORES = 16
NUM_WORKERS = NUM_CORES * NUM_SUBCORES
CHUNK = 128
NSLOTS = 6
OSLOTS = 2
SCALE = math.sqrt(D_MODEL)


def _sc_embed(xT, table, batch, hist):
    n_blocks = batch // CHUNK
    mesh = plsc.VectorSubcoreMesh(core_axis_name="c", subcore_axis_name="s")

    @functools.partial(
        pl.kernel,
        mesh=mesh,
        out_type=jax.ShapeDtypeStruct(
            (hist, 8, n_blocks, 8, 128), jnp.float32),
        scratch_types=[
            pltpu.VMEM((hist, CHUNK), jnp.int32),
            pltpu.VMEM((NSLOTS, CHUNK, D_MODEL), jnp.float32),
            pltpu.VMEM((OSLOTS, 8, 8, 129), jnp.float32),
            pltpu.SemaphoreType.DMA,
            pltpu.SemaphoreType.DMA,
        ],
        compiler_params=pltpu.CompilerParams(
            use_tc_tiling_on_sc=False, needs_layout_passes=False),
    )
    def k(xT_hbm, table_hbm, out_hbm, idx_v, rows_v, t5_v, gsem, osem):
        wid = lax.axis_index("s") * NUM_CORES + lax.axis_index("c")
        pltpu.sync_copy(xT_hbm.at[:, pl.ds(wid * CHUNK, CHUNK)], idx_v)

        def gather(h, slot):
            return pltpu.async_copy(
                table_hbm.at[idx_v.at[h]], rows_v.at[slot], gsem)

        for _p in range(4):
            gather(_p, _p)

        def transpose_scale(gslot, oslot):
            iota = lax.iota(jnp.int32, LANES)
            ti_half = jax.lax.shift_right_logical(iota, 3)
            r_vec = jax.lax.bitwise_and(iota, 7)
            dst = t5_v.at[oslot]

            @plsc.parallel_loop(0, CHUNK, unroll=8)
            def _c_loop(c):
                c16 = jnp.full((LANES,), 0, jnp.int32) + c
                for k in range(D_MODEL // LANES):
                    v = rows_v[gslot, c, pl.ds(k * LANES, LANES)]
                    plsc.store_scatter(
                        dst, [ti_half + (2 * k), r_vec, c16], v * SCALE)

        def body(h, carry):
            gslot = lax.rem(h, NSLOTS)
            oslot = lax.rem(h, OSLOTS)
            pltpu.make_async_copy(
                table_hbm.at[idx_v.at[h]], rows_v.at[gslot], gsem).wait()

            @pl.when(h >= OSLOTS)
            def _wait_out():
                pltpu.make_async_copy(
                    t5_v.at[oslot, :, :, pl.ds(0, 128)],
                    out_hbm.at[h, :, wid], osem).wait()

            transpose_scale(gslot, oslot)
            pltpu.async_copy(
                t5_v.at[oslot, :, :, pl.ds(0, 128)],
                out_hbm.at[h, :, wid], osem)

            @pl.when(h + 4 < hist)
            def _next_gather():
                gather(h + 4, lax.rem(h + 4, NSLOTS))
            return carry

        lax.fori_loop(0, hist, body, 0)

        for _ in range(OSLOTS):
            pltpu.make_async_copy(
                t5_v.at[0, :, :, pl.ds(0, 128)],
                out_hbm.at[0, :, wid], osem).wait()

    return k(xT, table)


def kernel(x, emb_weight):
    batch, hist = x.shape
    xT = x.astype(jnp.int32).T
    out5 = _sc_embed(xT, emb_weight, batch, hist)
    return out5.transpose(2, 4, 0, 1, 3).reshape(batch, hist, D_MODEL)
